# baseline (speedup 1.0000x reference)
import functools

import jax
import jax.numpy as jnp
from jax import lax
from jax.experimental import pallas as pl
from jax.experimental.pallas import tpu as pltpu
from jax.experimental.pallas import tpu_sc as plsc

_NUM_WORKERS = 32
_NBUF = 5
_LOOKAHEAD = 4


@functools.cache
def _make_gather(batch: int, seq: int, embed: int):
    assert batch % _NUM_WORKERS == 0 and _LOOKAHEAD < _NBUF
    assert seq % _NBUF == 0
    bpw = batch // _NUM_WORKERS

    mesh = plsc.VectorSubcoreMesh(core_axis_name="c", subcore_axis_name="s")

    @functools.partial(
        pl.kernel,
        mesh=mesh,
        out_type=jax.ShapeDtypeStruct((seq, batch, embed), jnp.float32),
        scratch_types=[
            pltpu.VMEM((seq, bpw), jnp.int32),
        ]
        + [pltpu.VMEM((bpw, embed), jnp.float32) for _ in range(_NBUF)]
        + [pltpu.SemaphoreType.DMA for _ in range(2 * _NBUF)],
    )
    def gather_kernel(idx_hbm, table_hbm, out_hbm, idx_v, *bufs_and_sems):
        rows = bufs_and_sems[:_NBUF]
        gsem = bufs_and_sems[_NBUF : 2 * _NBUF]
        wsem = bufs_and_sems[2 * _NBUF :]

        wid = lax.axis_index("s") * 2 + lax.axis_index("c")
        base = wid * bpw
        pltpu.sync_copy(idx_hbm.at[wid], idx_v)

        def gather_copy(j, slot):
            return pltpu.make_async_copy(
                table_hbm.at[idx_v.at[j]], rows[slot], gsem[slot]
            )

        def wb_copy(j, slot):
            return pltpu.make_async_copy(
                rows[slot], out_hbm.at[j, pl.ds(base, bpw)], wsem[slot]
            )

        for j in range(_LOOKAHEAD):
            gather_copy(j, j).start()

        def group(i, carry):
            for b in range(_NBUF):
                j = i * _NBUF + b
                gather_copy(j, b).wait()
                wb_copy(j, b).start()
                s2 = (b + _LOOKAHEAD) % _NBUF
                jn = j + _LOOKAHEAD

                @pl.when((jn >= _NBUF) & (jn < seq))
                def _():
                    wb_copy(jn - _NBUF, s2).wait()

                @pl.when(jn < seq)
                def _():
                    gather_copy(jn, s2).start()

            return carry

        lax.fori_loop(0, seq // _NBUF, group, 0)

        for b in range(_NBUF):
            wb_copy(seq - _NBUF + b, b).wait()

    return gather_kernel


def kernel(cleavage_indices, pos_embed):
    b, s = cleavage_indices.shape
    embed = pos_embed.shape[1]
    bpw = b // _NUM_WORKERS
    idx3 = (
        cleavage_indices.astype(jnp.int32)
        .reshape(_NUM_WORKERS, bpw, s)
        .transpose(0, 2, 1)
    )
    out = _make_gather(b, s, embed)(idx3, pos_embed)
    return out.transpose(1, 0, 2)


# speedup vs baseline: 1.3896x; 1.3896x over previous
import functools

import jax
import jax.numpy as jnp
from jax import lax
from jax.experimental import pallas as pl
from jax.experimental.pallas import tpu as pltpu
from jax.experimental.pallas import tpu_sc as plsc

_NUM_WORKERS = 32
_TILES = 16
_NBUF = 3
_LOOKAHEAD = 2


@functools.cache
def _make_gather(batch: int, seq: int, embed: int, table_rows: int):
    assert batch % _NUM_WORKERS == 0 and _LOOKAHEAD < _NBUF
    assert table_rows % (_TILES * 8) == 0
    bpw = batch // _NUM_WORKERS
    stage_blk = table_rows // _TILES

    mesh = plsc.VectorSubcoreMesh(core_axis_name="c", subcore_axis_name="s")

    @functools.partial(
        pl.kernel,
        mesh=mesh,
        out_type=jax.ShapeDtypeStruct((seq, batch, embed), jnp.float32),
        scratch_types=[
            pltpu.VMEM((seq, bpw), jnp.int32),
            pltpu.VMEM_SHARED((table_rows, embed), jnp.float32),
        ]
        + [pltpu.VMEM((bpw, embed), jnp.float32) for _ in range(_NBUF)]
        + [pltpu.SemaphoreType.DMA for _ in range(2 * _NBUF)],
    )
    def gather_kernel(idx_hbm, table_hbm, out_hbm, idx_v, table_sh, *bufs_and_sems):
        rows = bufs_and_sems[:_NBUF]
        gsem = bufs_and_sems[_NBUF : 2 * _NBUF]
        wsem = bufs_and_sems[2 * _NBUF :]

        sid = lax.axis_index("s")
        wid = sid * 2 + lax.axis_index("c")
        base = wid * bpw

        pltpu.sync_copy(
            table_hbm.at[pl.ds(sid * stage_blk, stage_blk)],
            table_sh.at[pl.ds(sid * stage_blk, stage_blk)],
        )
        pltpu.sync_copy(idx_hbm.at[wid], idx_v)
        plsc.subcore_barrier()

        def gather_copy(j, slot):
            return pltpu.make_async_copy(
                table_sh.at[idx_v.at[j]], rows[slot], gsem[slot]
            )

        def wb_copy(j, slot):
            return pltpu.make_async_copy(
                rows[slot], out_hbm.at[j, pl.ds(base, bpw)], wsem[slot]
            )

        for j in range(_LOOKAHEAD):
            gather_copy(j, j).start()

        def group(i, carry):
            for b in range(_NBUF):
                j = i * _NBUF + b

                @pl.when(j < seq)
                def _():
                    gather_copy(j, b).wait()
                    wb_copy(j, b).start()

                s2 = (b + _LOOKAHEAD) % _NBUF
                jn = j + _LOOKAHEAD

                @pl.when((jn >= _NBUF) & (jn < seq))
                def _():
                    wb_copy(jn - _NBUF, s2).wait()

                @pl.when(jn < seq)
                def _():
                    gather_copy(jn, s2).start()

            return carry

        lax.fori_loop(0, -(-seq // _NBUF), group, 0)

        for b in range(_NBUF):
            jj = seq - _NBUF + b
            wb_copy(jj, jj % _NBUF).wait()

    return gather_kernel


def kernel(cleavage_indices, pos_embed):
    b, s = cleavage_indices.shape
    embed = pos_embed.shape[1]
    bpw = b // _NUM_WORKERS
    idx3 = (
        cleavage_indices.astype(jnp.int32)
        .reshape(_NUM_WORKERS, bpw, s)
        .transpose(0, 2, 1)
    )
    out = _make_gather(b, s, embed, pos_embed.shape[0])(idx3, pos_embed)
    return out.transpose(1, 0, 2)


# speedup vs baseline: 1.5236x; 1.0965x over previous
import functools

import jax
import jax.numpy as jnp
from jax import lax
from jax.experimental import pallas as pl
from jax.experimental.pallas import tpu as pltpu
from jax.experimental.pallas import tpu_sc as plsc

_NUM_WORKERS = 32
_TILES = 16
_NBUF = 6
_LOOKAHEAD = 4
_CPS = 2


@functools.cache
def _make_gather(batch: int, seq: int, embed: int, table_rows: int):
    assert batch % _NUM_WORKERS == 0 and _LOOKAHEAD < _NBUF
    assert table_rows % (_TILES * 8) == 0
    bpw = batch // _NUM_WORKERS
    chunk = bpw // _CPS
    nchunks = seq * _CPS
    stage_blk = table_rows // _TILES

    mesh = plsc.VectorSubcoreMesh(core_axis_name="c", subcore_axis_name="s")

    @functools.partial(
        pl.kernel,
        mesh=mesh,
        out_type=jax.ShapeDtypeStruct((seq, batch, embed), jnp.float32),
        scratch_types=[
            pltpu.VMEM((seq, bpw), jnp.int32),
            pltpu.VMEM_SHARED((table_rows, embed), jnp.float32),
        ]
        + [pltpu.VMEM((chunk, embed), jnp.float32) for _ in range(_NBUF)]
        + [pltpu.SemaphoreType.DMA for _ in range(2 * _NBUF)],
    )
    def gather_kernel(idx_hbm, table_hbm, out_hbm, idx_v, table_sh, *bufs_and_sems):
        rows = bufs_and_sems[:_NBUF]
        gsem = bufs_and_sems[_NBUF : 2 * _NBUF]
        wsem = bufs_and_sems[2 * _NBUF :]

        sid = lax.axis_index("s")
        wid = sid * 2 + lax.axis_index("c")
        base = wid * bpw

        pltpu.sync_copy(
            table_hbm.at[pl.ds(sid * stage_blk, stage_blk)],
            table_sh.at[pl.ds(sid * stage_blk, stage_blk)],
        )
        pltpu.sync_copy(idx_hbm.at[wid], idx_v)
        plsc.subcore_barrier()

        def gather_copy(c, slot):
            return pltpu.make_async_copy(
                table_sh.at[idx_v.at[c // _CPS, pl.ds((c % _CPS) * chunk, chunk)]],
                rows[slot],
                gsem[slot],
            )

        def wb_copy(c, slot):
            return pltpu.make_async_copy(
                rows[slot],
                out_hbm.at[c // _CPS, pl.ds(base + (c % _CPS) * chunk, chunk)],
                wsem[slot],
            )

        for j in range(_LOOKAHEAD):
            gather_copy(j, j).start()

        def group(i, carry):
            for b in range(_NBUF):
                j = i * _NBUF + b

                @pl.when(j < nchunks)
                def _():
                    gather_copy(j, b).wait()
                    wb_copy(j, b).start()

                s2 = (b + _LOOKAHEAD) % _NBUF
                jn = j + _LOOKAHEAD

                @pl.when((jn >= _NBUF) & (jn < nchunks))
                def _():
                    wb_copy(jn - _NBUF, s2).wait()

                @pl.when(jn < nchunks)
                def _():
                    gather_copy(jn, s2).start()

            return carry

        lax.fori_loop(0, -(-nchunks // _NBUF), group, 0)

        for b in range(_NBUF):
            jj = nchunks - _NBUF + b
            wb_copy(jj, jj % _NBUF).wait()

    return gather_kernel


def kernel(cleavage_indices, pos_embed):
    b, s = cleavage_indices.shape
    embed = pos_embed.shape[1]
    bpw = b // _NUM_WORKERS
    idx3 = (
        cleavage_indices.astype(jnp.int32)
        .reshape(_NUM_WORKERS, bpw, s)
        .transpose(0, 2, 1)
    )
    out = _make_gather(b, s, embed, pos_embed.shape[0])(idx3, pos_embed)
    return out.transpose(1, 0, 2)


# speedup vs baseline: 1.5275x; 1.0025x over previous
import functools

import jax
import jax.numpy as jnp
from jax import lax
from jax.experimental import pallas as pl
from jax.experimental.pallas import tpu as pltpu
from jax.experimental.pallas import tpu_sc as plsc

_NUM_WORKERS = 32
_TILES = 16
_NBUF = 12
_LOOKAHEAD = 8
_CPS = 4


@functools.cache
def _make_gather(batch: int, seq: int, embed: int, table_rows: int):
    assert batch % _NUM_WORKERS == 0 and _LOOKAHEAD < _NBUF
    assert table_rows % (_TILES * 8) == 0
    bpw = batch // _NUM_WORKERS
    chunk = bpw // _CPS
    nchunks = seq * _CPS
    stage_blk = table_rows // _TILES

    mesh = plsc.VectorSubcoreMesh(core_axis_name="c", subcore_axis_name="s")

    @functools.partial(
        pl.kernel,
        mesh=mesh,
        out_type=jax.ShapeDtypeStruct((seq, batch, embed), jnp.float32),
        scratch_types=[
            pltpu.VMEM((seq, bpw), jnp.int32),
            pltpu.VMEM_SHARED((table_rows, embed), jnp.float32),
        ]
        + [pltpu.VMEM((chunk, embed), jnp.float32) for _ in range(_NBUF)]
        + [pltpu.SemaphoreType.DMA for _ in range(2 * _NBUF)],
    )
    def gather_kernel(idx_hbm, table_hbm, out_hbm, idx_v, table_sh, *bufs_and_sems):
        rows = bufs_and_sems[:_NBUF]
        gsem = bufs_and_sems[_NBUF : 2 * _NBUF]
        wsem = bufs_and_sems[2 * _NBUF :]

        sid = lax.axis_index("s")
        wid = sid * 2 + lax.axis_index("c")
        base = wid * bpw

        pltpu.sync_copy(
            table_hbm.at[pl.ds(sid * stage_blk, stage_blk)],
            table_sh.at[pl.ds(sid * stage_blk, stage_blk)],
        )
        pltpu.sync_copy(idx_hbm.at[wid], idx_v)
        plsc.subcore_barrier()

        def gather_copy(c, slot):
            return pltpu.make_async_copy(
                table_sh.at[idx_v.at[c // _CPS, pl.ds((c % _CPS) * chunk, chunk)]],
                rows[slot],
                gsem[slot],
            )

        def wb_copy(c, slot):
            return pltpu.make_async_copy(
                rows[slot],
                out_hbm.at[c // _CPS, pl.ds(base + (c % _CPS) * chunk, chunk)],
                wsem[slot],
            )

        for j in range(_LOOKAHEAD):
            gather_copy(j, j).start()

        def group(i, carry):
            for b in range(_NBUF):
                j = i * _NBUF + b

                @pl.when(j < nchunks)
                def _():
                    gather_copy(j, b).wait()
                    wb_copy(j, b).start()

                s2 = (b + _LOOKAHEAD) % _NBUF
                jn = j + _LOOKAHEAD

                @pl.when((jn >= _NBUF) & (jn < nchunks))
                def _():
                    wb_copy(jn - _NBUF, s2).wait()

                @pl.when(jn < nchunks)
                def _():
                    gather_copy(jn, s2).start()

            return carry

        lax.fori_loop(0, -(-nchunks // _NBUF), group, 0)

        for b in range(_NBUF):
            jj = nchunks - _NBUF + b
            wb_copy(jj, jj % _NBUF).wait()

    return gather_kernel


def kernel(cleavage_indices, pos_embed):
    b, s = cleavage_indices.shape
    embed = pos_embed.shape[1]
    bpw = b // _NUM_WORKERS
    idx3 = (
        cleavage_indices.astype(jnp.int32)
        .reshape(_NUM_WORKERS, bpw, s)
        .transpose(0, 2, 1)
    )
    out = _make_gather(b, s, embed, pos_embed.shape[0])(idx3, pos_embed)
    return out.transpose(1, 0, 2)
